# baseline (speedup 1.0000x reference)
import jax
import jax.numpy as jnp
from jax import lax
from jax.experimental import pallas as pl
from jax.experimental.pallas import tpu as pltpu
from jax.experimental.pallas import tpu_sc as plsc

NEG_SLOPE = 0.2
L = 16
NC = 2
NS = 16
NW = NC * NS
BE = 128
BH = 64
H = 128
DR = 88


def _hreduce16(v, op):
    vals = [v[i] for i in range(L)]
    while len(vals) > 1:
        vals = [op(vals[i], vals[i + 1]) for i in range(0, len(vals), 2)]
    return vals[0]


def _gmax_of(pmax_v):
    m = pmax_v[0, 0]
    for i in range(1, NW):
        m = jnp.maximum(m, pmax_v[i, 0])
    return _hreduce16(m, jnp.maximum)


def _matmul_call(x, w_cat, n, in_ch):
    bm = 512
    grid = (pl.cdiv(n, bm),)

    def mm_kernel(x_ref, w_ref, olf, orf, o0, o1):
        res = jax.lax.dot_general(
            x_ref[...], w_ref[...], (((1,), (0,)), ((), ())),
            preferred_element_type=jnp.float32)
        olf[...] = res[:, :2 * H]
        orf[...] = res[:, 2 * H:]
        o0[...] = res[:, 0 * H:1 * H]
        o1[...] = res[:, 1 * H:2 * H]

    return pl.pallas_call(
        mm_kernel,
        grid=grid,
        in_specs=[
            pl.BlockSpec((bm, in_ch), lambda i: (i, 0)),
            pl.BlockSpec((in_ch, 4 * H), lambda i: (0, 0)),
        ],
        out_specs=[pl.BlockSpec((bm, 2 * H), lambda i: (i, 0)),
                   pl.BlockSpec((bm, 2 * H), lambda i: (i, 0)),
                   pl.BlockSpec((bm, H), lambda i: (i, 0)),
                   pl.BlockSpec((bm, H), lambda i: (i, 0))],
        out_shape=[jax.ShapeDtypeStruct((n, 2 * H), jnp.float32),
                   jax.ShapeDtypeStruct((n, 2 * H), jnp.float32),
                   jax.ShapeDtypeStruct((n, H), jnp.float32),
                   jax.ShapeDtypeStruct((n, H), jnp.float32)],
    )(x, w_cat)


def _alpha_call(xlf, xrf, src2d, dst2d, att, e_tot):
    nrows = src2d.shape[0]
    t_per_w = nrows // NW
    hc = H // L

    def body(xlf_h, xrf_h, src_h, dst_h, att_h,
             alpha_h, pmax_h,
             src_v, dst_v, alpha_v, att_v,
             ua, va, ub, vb, maxv, semA, semB):
        wid = lax.axis_index("s") * NC + lax.axis_index("c")
        rbase = wid * t_per_w
        pltpu.sync_copy(src_h.at[pl.ds(rbase, t_per_w)], src_v)
        pltpu.sync_copy(dst_h.at[pl.ds(rbase, t_per_w)], dst_v)
        pltpu.sync_copy(att_h, att_v)
        lane = lax.iota(jnp.int32, L)
        attv = [att_v[pl.ds(c * L, L)] for c in range(2 * hc)]

        def start(t, half, bu, bv, sem):
            s_sl = src_v.at[t, 0, pl.ds(half * BH, BH)]
            d_sl = dst_v.at[t, 0, pl.ds(half * BH, BH)]
            pltpu.async_copy(xlf_h.at[s_sl], bu, sem)
            pltpu.async_copy(xrf_h.at[d_sl], bv, sem)

        def drain(bu, bv, sem):
            pltpu.make_async_copy(xlf_h.at[src_v.at[0, 0, pl.ds(0, BH)]],
                                  bu, sem).wait()
            pltpu.make_async_copy(xrf_h.at[dst_v.at[0, 0, pl.ds(0, BH)]],
                                  bv, sem).wait()

        def compute(t, half, bu, bv, mx):
            ebase = (rbase + t) * BE + half * BH

            def eblock(eg, mx2):
                a_vec = jnp.zeros((L,), jnp.float32)
                for j in range(L):
                    e = eg * L + j
                    acc0 = jnp.zeros((L,), jnp.float32)
                    acc1 = jnp.zeros((L,), jnp.float32)
                    for c in range(2 * hc):
                        z = bu[e, pl.ds(c * L, L)] + bv[e, pl.ds(c * L, L)]
                        lr = jnp.maximum(z, NEG_SLOPE * z)
                        if c % 2 == 0:
                            acc0 = acc0 + lr * attv[c]
                        else:
                            acc1 = acc1 + lr * attv[c]
                    sj = _hreduce16(acc0 + acc1, jnp.add)
                    a_vec = jnp.where(lane == j, sj, a_vec)
                gid = ebase + eg * L + lane
                a_vec = jnp.where(gid < e_tot, a_vec, jnp.float32(-1e30))
                alpha_v[t, 0, pl.ds(half * BH + eg * L, L)] = a_vec
                return jnp.maximum(mx2, a_vec)

            return lax.fori_loop(0, BH // L, eblock, mx)

        start(0, 0, ua, va, semA)
        start(0, 1, ub, vb, semB)

        def rowstep(t, mx):
            drain(ua, va, semA)
            mx = compute(t, 0, ua, va, mx)

            @pl.when(t < t_per_w - 1)
            def _():
                start(t + 1, 0, ua, va, semA)
            drain(ub, vb, semB)
            mx = compute(t, 1, ub, vb, mx)

            @pl.when(t < t_per_w - 1)
            def _():
                start(t + 1, 1, ub, vb, semB)
            return mx

        mvec = lax.fori_loop(0, t_per_w, rowstep,
                             jnp.full((L,), -1e30, jnp.float32))
        maxv[0] = jnp.full((L,), _hreduce16(mvec, jnp.maximum), jnp.float32)
        pltpu.sync_copy(alpha_v, alpha_h.at[pl.ds(rbase, t_per_w)])
        pltpu.sync_copy(maxv, pmax_h.at[wid])

    gb = lambda: pltpu.VMEM((BH, 2 * H), jnp.float32)
    fn = pl.kernel(
        body,
        out_type=(jax.ShapeDtypeStruct((nrows, 1, BE), jnp.float32),
                  jax.ShapeDtypeStruct((NW, 1, L), jnp.float32)),
        mesh=plsc.VectorSubcoreMesh(core_axis_name="c", subcore_axis_name="s"),
        scratch_types=[
            pltpu.VMEM((t_per_w, 1, BE), jnp.int32),
            pltpu.VMEM((t_per_w, 1, BE), jnp.int32),
            pltpu.VMEM((t_per_w, 1, BE), jnp.float32),
            pltpu.VMEM((2 * H,), jnp.float32),
            gb(), gb(), gb(), gb(),
            pltpu.VMEM((1, L), jnp.float32),
            pltpu.SemaphoreType.DMA,
            pltpu.SemaphoreType.DMA,
        ],
    )
    return fn(xlf, xrf, src2d, dst2d, att)


def _denom_call(dst2d, alpha2d, pmax):
    nrows = dst2d.shape[0]
    t_per_w = nrows // NW

    def body(dst_h, alpha_h, pmax_h, den_h,
             dst_v, alpha_v, pmax_v, mden, didx, dtmp, den_sh, sem):
        h = lax.axis_index("c")
        tid = lax.axis_index("s")
        wid = tid * NC + h
        rbase = wid * t_per_w
        pltpu.sync_copy(dst_h.at[pl.ds(rbase, t_per_w)], dst_v)
        pltpu.sync_copy(alpha_h.at[pl.ds(rbase, t_per_w)], alpha_v)
        pltpu.sync_copy(pmax_h, pmax_v)
        gmax = _gmax_of(pmax_v)
        lane = lax.iota(jnp.int32, L)
        lanes_c = [lane + c * L for c in range(H // L)]

        def zmb(i, _):
            for c in range(H // L):
                mden[i, pl.ds(c * L, L)] = jnp.zeros((L,), jnp.float32)
            return 0
        lax.fori_loop(0, BE, zmb, 0)

        @pl.when(tid < DR // 8)
        def _():
            pltpu.sync_copy(mden.at[pl.ds(0, 8)],
                            den_sh.at[pl.ds(tid * 8, 8)])
        plsc.subcore_barrier()

        def step(t, _):
            def egroup(k, _):
                a = alpha_v[t, 0, pl.ds(k * L, L)]
                wvec = jnp.exp(a - gmax)
                dvec = dst_v[t, 0, pl.ds(k * L, L)]
                dmod = jnp.bitwise_and(dvec, jnp.int32(H - 1))
                didx[0, pl.ds(k * L, L)] = jnp.right_shift(dvec, 7)
                e0 = k * L
                for j in range(L):
                    w = wvec[j]
                    dm = dmod[j]
                    for c in range(H // L):
                        mden[e0 + j, pl.ds(c * L, L)] = jnp.where(
                            lanes_c[c] == dm, w, jnp.float32(0.0))
                return 0
            lax.fori_loop(0, BE // L, egroup, 0)
            pltpu.sync_copy(mden, den_sh.at[didx.at[0]], add=True)
            return 0
        lax.fori_loop(0, t_per_w, step, 0)
        plsc.subcore_barrier()

        @pl.when(tid < DR // 8)
        def _():
            pltpu.sync_copy(den_sh.at[pl.ds(tid * 8, 8)], dtmp)
            pltpu.sync_copy(dtmp, den_h.at[h, pl.ds(tid * 8, 8)])

    fn = pl.kernel(
        body,
        out_type=jax.ShapeDtypeStruct((NC, DR, H), jnp.float32),
        mesh=plsc.VectorSubcoreMesh(core_axis_name="c", subcore_axis_name="s"),
        scratch_types=[
            pltpu.VMEM((t_per_w, 1, BE), jnp.int32),
            pltpu.VMEM((t_per_w, 1, BE), jnp.float32),
            pltpu.VMEM((NW, 1, L), jnp.float32),
            pltpu.VMEM((BE, H), jnp.float32),
            pltpu.VMEM((1, BE), jnp.int32),
            pltpu.VMEM((8, H), jnp.float32),
            pltpu.VMEM_SHARED((DR, H), jnp.float32),
            pltpu.SemaphoreType.DMA,
        ],
    )
    return fn(dst2d, alpha2d, pmax)


def _agg_call(xl0, xl1, src2d, dst2d, alpha2d, pmax, den2, bias, n):
    nrows = src2d.shape[0]
    t_rows = nrows // NS
    pairs = t_rows // 2
    npad = ((n + NS * L - 1) // (NS * L)) * (NS * L)
    npt = npad // NS
    hc = H // L
    drows = npt // H

    def body(xl0_h, xl1_h, src_h, dst_h, alpha_h, pmax_h, den_h, bias_h,
             out_h,
             src_s, dst_s, alpha_s, src_b, dst_b, alpha_b, pmax_v, bias_v,
             gA, gB, mbuf, dloc, dtmp, ebuf, obuf, out_sh, gsA, gsB):
        h = lax.axis_index("c")
        tid = lax.axis_index("s")
        rbase = tid * t_rows
        pltpu.sync_copy(pmax_h, pmax_v)
        pltpu.sync_copy(bias_h, bias_v)
        gmax = _gmax_of(pmax_v)

        def zmb(i, _):
            for c in range(hc):
                mbuf[i, pl.ds(c * L, L)] = jnp.zeros((L,), jnp.float32)
            return 0
        lax.fori_loop(0, BH, zmb, 0)
        n0 = tid * npt
        for b in range(npt // BH):
            pltpu.sync_copy(mbuf, out_sh.at[pl.ds(n0 + b * BH, BH)])
        plsc.subcore_barrier()

        def load_idx(t, s_r, d_r, a_r):
            pltpu.sync_copy(src_h.at[pl.ds(rbase + t, 1)], s_r)
            pltpu.sync_copy(dst_h.at[pl.ds(rbase + t, 1)], d_r)
            pltpu.sync_copy(alpha_h.at[pl.ds(rbase + t, 1)], a_r)

        def startg(s_r, half, buf, sem):
            s_sl = s_r.at[0, 0, pl.ds(half * BH, BH)]

            @pl.when(h == 0)
            def _():
                pltpu.async_copy(xl0_h.at[s_sl], buf, sem)

            @pl.when(h == 1)
            def _():
                pltpu.async_copy(xl1_h.at[s_sl], buf, sem)

        def draing(buf, sem):
            pltpu.make_async_copy(
                xl0_h.at[src_s.at[0, 0, pl.ds(0, BH)]], buf, sem).wait()

        def subblock(a_r, d_r, half, gbuf):
            def egroup(k, _):
                a = a_r[0, 0, pl.ds(half * BH + k * L, L)]
                wvec = jnp.exp(a - gmax)
                e0 = k * L
                for j in range(L):
                    w = wvec[j]
                    for c in range(hc):
                        mbuf[e0 + j, pl.ds(c * L, L)] = (
                            gbuf[e0 + j, pl.ds(c * L, L)] * w)
                return 0
            lax.fori_loop(0, BH // L, egroup, 0)
            d_sl = d_r.at[0, 0, pl.ds(half * BH, BH)]
            pltpu.sync_copy(mbuf, out_sh.at[d_sl], add=True)

        load_idx(0, src_s, dst_s, alpha_s)
        startg(src_s, 0, gA, gsA)

        def pairstep(p, _):
            load_idx(2 * p + 1, src_b, dst_b, alpha_b)
            draing(gA, gsA)
            startg(src_s, 1, gB, gsB)
            subblock(alpha_s, dst_s, 0, gA)
            draing(gB, gsB)
            startg(src_b, 0, gA, gsA)
            subblock(alpha_s, dst_s, 1, gB)
            draing(gA, gsA)
            startg(src_b, 1, gB, gsB)
            subblock(alpha_b, dst_b, 0, gA)
            draing(gB, gsB)

            @pl.when(p < pairs - 1)
            def _():
                load_idx(2 * p + 2, src_s, dst_s, alpha_s)
                startg(src_s, 0, gA, gsA)
            subblock(alpha_b, dst_b, 1, gB)
            return 0
        lax.fori_loop(0, pairs, pairstep, 0)
        plsc.subcore_barrier()

        r_lo = tid * drows
        w0 = (r_lo // 8) * 8
        pltpu.sync_copy(den_h.at[0, pl.ds(w0, 2 * 8)], dloc)
        pltpu.sync_copy(den_h.at[1, pl.ds(w0, 2 * 8)], dtmp)
        roff = r_lo - w0

        nvalid = jnp.maximum(0, jnp.minimum(npt, n - n0))
        nblk = nvalid // L
        bbs = [bias_v[pl.ds(h * H + c * L, L)] for c in range(hc)]

        def blk(b, _):
            r0 = n0 + b * L
            pltpu.sync_copy(out_sh.at[pl.ds(r0, L)], ebuf)
            drow = roff + b // hc
            dsl = pl.ds((b % hc) * L, L)
            dv = dloc[drow, dsl] + dtmp[drow, dsl]
            ivec = jnp.float32(1.0) / (dv + jnp.float32(1e-16))
            for j in range(L):
                sj = ivec[j]
                for c in range(hc):
                    obuf[j, pl.ds(c * L, L)] = (
                        ebuf[j, pl.ds(c * L, L)] * sj + bbs[c])
            pltpu.sync_copy(obuf, out_h.at[h, pl.ds(r0, L)])
            return 0
        lax.fori_loop(0, nblk, blk, 0)

    fn = pl.kernel(
        body,
        out_type=jax.ShapeDtypeStruct((NC, n, H), jnp.float32),
        mesh=plsc.VectorSubcoreMesh(core_axis_name="c", subcore_axis_name="s"),
        scratch_types=[
            pltpu.VMEM((1, 1, BE), jnp.int32),
            pltpu.VMEM((1, 1, BE), jnp.int32),
            pltpu.VMEM((1, 1, BE), jnp.float32),
            pltpu.VMEM((1, 1, BE), jnp.int32),
            pltpu.VMEM((1, 1, BE), jnp.int32),
            pltpu.VMEM((1, 1, BE), jnp.float32),
            pltpu.VMEM((NW, 1, L), jnp.float32),
            pltpu.VMEM((2 * H,), jnp.float32),
            pltpu.VMEM((BH, H), jnp.float32),
            pltpu.VMEM((BH, H), jnp.float32),
            pltpu.VMEM((BH, H), jnp.float32),
            pltpu.VMEM((2 * 8, H), jnp.float32),
            pltpu.VMEM((2 * 8, H), jnp.float32),
            pltpu.VMEM((L, H), jnp.float32),
            pltpu.VMEM((L, H), jnp.float32),
            pltpu.VMEM_SHARED((npad, H), jnp.float32),
            pltpu.SemaphoreType.DMA,
            pltpu.SemaphoreType.DMA,
        ],
    )
    return fn(xl0, xl1, src2d, dst2d, alpha2d, pmax, den2, bias)


def kernel(x, edge_index, W_l, W_r, att, bias):
    n, in_ch = x.shape
    e = edge_index.shape[1]
    e_tot = e + n
    loop = jnp.arange(n, dtype=edge_index.dtype)
    src = jnp.concatenate([edge_index[0], loop])
    dst = jnp.concatenate([edge_index[1], loop])
    rows_per_w = pl.cdiv(e_tot, NW * BE)
    ep = NW * rows_per_w * BE
    pad = ep - e_tot
    src2d = jnp.concatenate([src, jnp.zeros((pad,), src.dtype)]).reshape(
        ep // BE, 1, BE)
    dst2d = jnp.concatenate([dst, jnp.zeros((pad,), dst.dtype)]).reshape(
        ep // BE, 1, BE)
    w_cat = jnp.concatenate([W_l, W_r], axis=1)

    xlf, xrf, xl0, xl1 = _matmul_call(x, w_cat, n, in_ch)
    alpha2d, pmax = _alpha_call(xlf, xrf, src2d, dst2d, att, e_tot)
    den2 = _denom_call(dst2d, alpha2d, pmax)
    out2 = _agg_call(xl0, xl1, src2d, dst2d, alpha2d, pmax, den2, bias, n)
    return jnp.concatenate([out2[0], out2[1]], axis=1)


# speedup vs baseline: 1.0319x; 1.0319x over previous
import jax
import jax.numpy as jnp
from jax import lax
from jax.experimental import pallas as pl
from jax.experimental.pallas import tpu as pltpu
from jax.experimental.pallas import tpu_sc as plsc

NEG_SLOPE = 0.2
L = 16
NC = 2
NS = 16
NW = NC * NS
BE = 128
BH = 64
H = 128
DR = 88


def _hreduce16(v, op):
    vals = [v[i] for i in range(L)]
    while len(vals) > 1:
        vals = [op(vals[i], vals[i + 1]) for i in range(0, len(vals), 2)]
    return vals[0]


def _gmax_of(pmax_v):
    m = pmax_v[0, 0]
    for i in range(1, NW):
        m = jnp.maximum(m, pmax_v[i, 0])
    return _hreduce16(m, jnp.maximum)


def _matmul_call(x, w_cat, n, in_ch):
    bm = 512
    grid = (pl.cdiv(n, bm),)

    def mm_kernel(x_ref, w_ref, o0, o1, o2, o3):
        res = jax.lax.dot_general(
            x_ref[...], w_ref[...], (((1,), (0,)), ((), ())),
            preferred_element_type=jnp.float32)
        o0[...] = res[:, 0 * H:1 * H]
        o1[...] = res[:, 1 * H:2 * H]
        o2[...] = res[:, 2 * H:3 * H]
        o3[...] = res[:, 3 * H:4 * H]

    return pl.pallas_call(
        mm_kernel,
        grid=grid,
        in_specs=[
            pl.BlockSpec((bm, in_ch), lambda i: (i, 0)),
            pl.BlockSpec((in_ch, 4 * H), lambda i: (0, 0)),
        ],
        out_specs=[pl.BlockSpec((bm, H), lambda i: (i, 0))] * 4,
        out_shape=[jax.ShapeDtypeStruct((n, H), jnp.float32)] * 4,
    )(x, w_cat)


def _alpha_call(xl0, xl1, xr0, xr1, src2d, dst2d, att, e_tot):
    nrows = src2d.shape[0]
    t_per_w = nrows // NW
    hc = H // L

    def body(xl0_h, xl1_h, xr0_h, xr1_h, src_h, dst_h, att_h,
             alpha_h, pmax_h,
             src_v, dst_v, alpha_v, att_v,
             ua0, ua1, va0, va1, ub0, ub1, vb0, vb1, maxv, semA, semB):
        wid = lax.axis_index("s") * NC + lax.axis_index("c")
        rbase = wid * t_per_w
        pltpu.sync_copy(src_h.at[pl.ds(rbase, t_per_w)], src_v)
        pltpu.sync_copy(dst_h.at[pl.ds(rbase, t_per_w)], dst_v)
        pltpu.sync_copy(att_h, att_v)
        lane = lax.iota(jnp.int32, L)
        attv = [att_v[pl.ds(c * L, L)] for c in range(2 * hc)]

        def start(t, half, b0, b1, b2, b3, sem):
            s_sl = src_v.at[t, 0, pl.ds(half * BH, BH)]
            d_sl = dst_v.at[t, 0, pl.ds(half * BH, BH)]
            pltpu.async_copy(xl0_h.at[s_sl], b0, sem)
            pltpu.async_copy(xl1_h.at[s_sl], b1, sem)
            pltpu.async_copy(xr0_h.at[d_sl], b2, sem)
            pltpu.async_copy(xr1_h.at[d_sl], b3, sem)

        def drain(b0, b1, b2, b3, sem):
            pltpu.make_async_copy(xl0_h.at[src_v.at[0, 0, pl.ds(0, BH)]],
                                  b0, sem).wait()
            pltpu.make_async_copy(xl1_h.at[src_v.at[0, 0, pl.ds(0, BH)]],
                                  b1, sem).wait()
            pltpu.make_async_copy(xr0_h.at[dst_v.at[0, 0, pl.ds(0, BH)]],
                                  b2, sem).wait()
            pltpu.make_async_copy(xr1_h.at[dst_v.at[0, 0, pl.ds(0, BH)]],
                                  b3, sem).wait()

        def compute(t, half, u0, u1, v0, v1, mx):
            ebase = (rbase + t) * BE + half * BH

            def eblock(eg, mx2):
                a_vec = jnp.zeros((L,), jnp.float32)
                for j in range(L):
                    e = eg * L + j
                    acc0 = jnp.zeros((L,), jnp.float32)
                    acc1 = jnp.zeros((L,), jnp.float32)
                    for c in range(hc):
                        z = u0[e, pl.ds(c * L, L)] + v0[e, pl.ds(c * L, L)]
                        lr = jnp.maximum(z, NEG_SLOPE * z)
                        if c % 2 == 0:
                            acc0 = acc0 + lr * attv[c]
                        else:
                            acc1 = acc1 + lr * attv[c]
                    for c in range(hc):
                        z = u1[e, pl.ds(c * L, L)] + v1[e, pl.ds(c * L, L)]
                        lr = jnp.maximum(z, NEG_SLOPE * z)
                        if c % 2 == 0:
                            acc0 = acc0 + lr * attv[hc + c]
                        else:
                            acc1 = acc1 + lr * attv[hc + c]
                    sj = _hreduce16(acc0 + acc1, jnp.add)
                    a_vec = jnp.where(lane == j, sj, a_vec)
                gid = ebase + eg * L + lane
                a_vec = jnp.where(gid < e_tot, a_vec, jnp.float32(-1e30))
                alpha_v[t, 0, pl.ds(half * BH + eg * L, L)] = a_vec
                return jnp.maximum(mx2, a_vec)

            return lax.fori_loop(0, BH // L, eblock, mx)

        start(0, 0, ua0, ua1, va0, va1, semA)
        start(0, 1, ub0, ub1, vb0, vb1, semB)

        def rowstep(t, mx):
            drain(ua0, ua1, va0, va1, semA)
            mx = compute(t, 0, ua0, ua1, va0, va1, mx)

            @pl.when(t < t_per_w - 1)
            def _():
                start(t + 1, 0, ua0, ua1, va0, va1, semA)
            drain(ub0, ub1, vb0, vb1, semB)
            mx = compute(t, 1, ub0, ub1, vb0, vb1, mx)

            @pl.when(t < t_per_w - 1)
            def _():
                start(t + 1, 1, ub0, ub1, vb0, vb1, semB)
            return mx

        mvec = lax.fori_loop(0, t_per_w, rowstep,
                             jnp.full((L,), -1e30, jnp.float32))
        maxv[0] = jnp.full((L,), _hreduce16(mvec, jnp.maximum), jnp.float32)
        pltpu.sync_copy(alpha_v, alpha_h.at[pl.ds(rbase, t_per_w)])
        pltpu.sync_copy(maxv, pmax_h.at[wid])

    gb = lambda: pltpu.VMEM((BH, H), jnp.float32)
    fn = pl.kernel(
        body,
        out_type=(jax.ShapeDtypeStruct((nrows, 1, BE), jnp.float32),
                  jax.ShapeDtypeStruct((NW, 1, L), jnp.float32)),
        mesh=plsc.VectorSubcoreMesh(core_axis_name="c", subcore_axis_name="s"),
        scratch_types=[
            pltpu.VMEM((t_per_w, 1, BE), jnp.int32),
            pltpu.VMEM((t_per_w, 1, BE), jnp.int32),
            pltpu.VMEM((t_per_w, 1, BE), jnp.float32),
            pltpu.VMEM((2 * H,), jnp.float32),
            gb(), gb(), gb(), gb(), gb(), gb(), gb(), gb(),
            pltpu.VMEM((1, L), jnp.float32),
            pltpu.SemaphoreType.DMA,
            pltpu.SemaphoreType.DMA,
        ],
    )
    return fn(xl0, xl1, xr0, xr1, src2d, dst2d, att)


def _denom_call(dst2d, alpha2d, pmax):
    nrows = dst2d.shape[0]
    t_per_w = nrows // NW

    def body(dst_h, alpha_h, pmax_h, den_h,
             dst_v, alpha_v, pmax_v, mden, didx, dtmp, den_sh, sem):
        h = lax.axis_index("c")
        tid = lax.axis_index("s")
        wid = tid * NC + h
        rbase = wid * t_per_w
        pltpu.sync_copy(dst_h.at[pl.ds(rbase, t_per_w)], dst_v)
        pltpu.sync_copy(alpha_h.at[pl.ds(rbase, t_per_w)], alpha_v)
        pltpu.sync_copy(pmax_h, pmax_v)
        gmax = _gmax_of(pmax_v)
        lane = lax.iota(jnp.int32, L)
        lanes_c = [lane + c * L for c in range(H // L)]

        def zmb(i, _):
            for c in range(H // L):
                mden[i, pl.ds(c * L, L)] = jnp.zeros((L,), jnp.float32)
            return 0
        lax.fori_loop(0, BE, zmb, 0)

        @pl.when(tid < DR // 8)
        def _():
            pltpu.sync_copy(mden.at[pl.ds(0, 8)],
                            den_sh.at[pl.ds(tid * 8, 8)])
        plsc.subcore_barrier()

        def step(t, _):
            def egroup(k, _):
                a = alpha_v[t, 0, pl.ds(k * L, L)]
                wvec = jnp.exp(a - gmax)
                dvec = dst_v[t, 0, pl.ds(k * L, L)]
                dmod = jnp.bitwise_and(dvec, jnp.int32(H - 1))
                didx[0, pl.ds(k * L, L)] = jnp.right_shift(dvec, 7)
                e0 = k * L
                for j in range(L):
                    w = wvec[j]
                    dm = dmod[j]
                    for c in range(H // L):
                        mden[e0 + j, pl.ds(c * L, L)] = jnp.where(
                            lanes_c[c] == dm, w, jnp.float32(0.0))
                return 0
            lax.fori_loop(0, BE // L, egroup, 0)
            pltpu.sync_copy(mden, den_sh.at[didx.at[0]], add=True)
            return 0
        lax.fori_loop(0, t_per_w, step, 0)
        plsc.subcore_barrier()

        @pl.when(tid < DR // 8)
        def _():
            pltpu.sync_copy(den_sh.at[pl.ds(tid * 8, 8)], dtmp)
            pltpu.sync_copy(dtmp, den_h.at[h, pl.ds(tid * 8, 8)])

    fn = pl.kernel(
        body,
        out_type=jax.ShapeDtypeStruct((NC, DR, H), jnp.float32),
        mesh=plsc.VectorSubcoreMesh(core_axis_name="c", subcore_axis_name="s"),
        scratch_types=[
            pltpu.VMEM((t_per_w, 1, BE), jnp.int32),
            pltpu.VMEM((t_per_w, 1, BE), jnp.float32),
            pltpu.VMEM((NW, 1, L), jnp.float32),
            pltpu.VMEM((BE, H), jnp.float32),
            pltpu.VMEM((1, BE), jnp.int32),
            pltpu.VMEM((8, H), jnp.float32),
            pltpu.VMEM_SHARED((DR, H), jnp.float32),
            pltpu.SemaphoreType.DMA,
        ],
    )
    return fn(dst2d, alpha2d, pmax)


def _agg_call(xl0, xl1, src2d, dst2d, alpha2d, pmax, den2, bias, n):
    nrows = src2d.shape[0]
    t_rows = nrows // NS
    pairs = t_rows // 2
    npad = ((n + NS * L - 1) // (NS * L)) * (NS * L)
    npt = npad // NS
    hc = H // L
    drows = npt // H

    def body(xl0_h, xl1_h, src_h, dst_h, alpha_h, pmax_h, den_h, bias_h,
             out_h,
             src_s, dst_s, alpha_s, src_b, dst_b, alpha_b, pmax_v, bias_v,
             gA, gB, mbuf, dloc, dtmp, ebuf, obuf, out_sh, gsA, gsB):
        h = lax.axis_index("c")
        tid = lax.axis_index("s")
        rbase = tid * t_rows
        pltpu.sync_copy(pmax_h, pmax_v)
        pltpu.sync_copy(bias_h, bias_v)
        gmax = _gmax_of(pmax_v)

        def zmb(i, _):
            for c in range(hc):
                mbuf[i, pl.ds(c * L, L)] = jnp.zeros((L,), jnp.float32)
            return 0
        lax.fori_loop(0, BH, zmb, 0)
        n0 = tid * npt
        for b in range(npt // BH):
            pltpu.sync_copy(mbuf, out_sh.at[pl.ds(n0 + b * BH, BH)])
        plsc.subcore_barrier()

        def load_idx(t, s_r, d_r, a_r):
            pltpu.sync_copy(src_h.at[pl.ds(rbase + t, 1)], s_r)
            pltpu.sync_copy(dst_h.at[pl.ds(rbase + t, 1)], d_r)
            pltpu.sync_copy(alpha_h.at[pl.ds(rbase + t, 1)], a_r)

        def startg(s_r, half, buf, sem):
            s_sl = s_r.at[0, 0, pl.ds(half * BH, BH)]

            @pl.when(h == 0)
            def _():
                pltpu.async_copy(xl0_h.at[s_sl], buf, sem)

            @pl.when(h == 1)
            def _():
                pltpu.async_copy(xl1_h.at[s_sl], buf, sem)

        def draing(buf, sem):
            pltpu.make_async_copy(
                xl0_h.at[src_s.at[0, 0, pl.ds(0, BH)]], buf, sem).wait()

        def subblock(a_r, d_r, half, gbuf):
            def egroup(k, _):
                a = a_r[0, 0, pl.ds(half * BH + k * L, L)]
                wvec = jnp.exp(a - gmax)
                e0 = k * L
                for j in range(L):
                    w = wvec[j]
                    for c in range(hc):
                        mbuf[e0 + j, pl.ds(c * L, L)] = (
                            gbuf[e0 + j, pl.ds(c * L, L)] * w)
                return 0
            lax.fori_loop(0, BH // L, egroup, 0)
            d_sl = d_r.at[0, 0, pl.ds(half * BH, BH)]
            pltpu.sync_copy(mbuf, out_sh.at[d_sl], add=True)

        load_idx(0, src_s, dst_s, alpha_s)
        startg(src_s, 0, gA, gsA)

        def pairstep(p, _):
            load_idx(2 * p + 1, src_b, dst_b, alpha_b)
            draing(gA, gsA)
            startg(src_s, 1, gB, gsB)
            subblock(alpha_s, dst_s, 0, gA)
            draing(gB, gsB)
            startg(src_b, 0, gA, gsA)
            subblock(alpha_s, dst_s, 1, gB)
            draing(gA, gsA)
            startg(src_b, 1, gB, gsB)
            subblock(alpha_b, dst_b, 0, gA)
            draing(gB, gsB)

            @pl.when(p < pairs - 1)
            def _():
                load_idx(2 * p + 2, src_s, dst_s, alpha_s)
                startg(src_s, 0, gA, gsA)
            subblock(alpha_b, dst_b, 1, gB)
            return 0
        lax.fori_loop(0, pairs, pairstep, 0)
        plsc.subcore_barrier()

        r_lo = tid * drows
        w0 = (r_lo // 8) * 8
        pltpu.sync_copy(den_h.at[0, pl.ds(w0, 2 * 8)], dloc)
        pltpu.sync_copy(den_h.at[1, pl.ds(w0, 2 * 8)], dtmp)
        roff = r_lo - w0

        nvalid = jnp.maximum(0, jnp.minimum(npt, n - n0))
        nblk = nvalid // L
        bbs = [bias_v[pl.ds(h * H + c * L, L)] for c in range(hc)]

        def blk(b, _):
            r0 = n0 + b * L
            pltpu.sync_copy(out_sh.at[pl.ds(r0, L)], ebuf)
            drow = roff + b // hc
            dsl = pl.ds((b % hc) * L, L)
            dv = dloc[drow, dsl] + dtmp[drow, dsl]
            ivec = jnp.float32(1.0) / (dv + jnp.float32(1e-16))
            for j in range(L):
                sj = ivec[j]
                for c in range(hc):
                    obuf[j, pl.ds(c * L, L)] = (
                        ebuf[j, pl.ds(c * L, L)] * sj + bbs[c])
            pltpu.sync_copy(obuf, out_h.at[h, pl.ds(r0, L)])
            return 0
        lax.fori_loop(0, nblk, blk, 0)

    fn = pl.kernel(
        body,
        out_type=jax.ShapeDtypeStruct((NC, n, H), jnp.float32),
        mesh=plsc.VectorSubcoreMesh(core_axis_name="c", subcore_axis_name="s"),
        scratch_types=[
            pltpu.VMEM((1, 1, BE), jnp.int32),
            pltpu.VMEM((1, 1, BE), jnp.int32),
            pltpu.VMEM((1, 1, BE), jnp.float32),
            pltpu.VMEM((1, 1, BE), jnp.int32),
            pltpu.VMEM((1, 1, BE), jnp.int32),
            pltpu.VMEM((1, 1, BE), jnp.float32),
            pltpu.VMEM((NW, 1, L), jnp.float32),
            pltpu.VMEM((2 * H,), jnp.float32),
            pltpu.VMEM((BH, H), jnp.float32),
            pltpu.VMEM((BH, H), jnp.float32),
            pltpu.VMEM((BH, H), jnp.float32),
            pltpu.VMEM((2 * 8, H), jnp.float32),
            pltpu.VMEM((2 * 8, H), jnp.float32),
            pltpu.VMEM((L, H), jnp.float32),
            pltpu.VMEM((L, H), jnp.float32),
            pltpu.VMEM_SHARED((npad, H), jnp.float32),
            pltpu.SemaphoreType.DMA,
            pltpu.SemaphoreType.DMA,
        ],
    )
    return fn(xl0, xl1, src2d, dst2d, alpha2d, pmax, den2, bias)


def kernel(x, edge_index, W_l, W_r, att, bias):
    n, in_ch = x.shape
    e = edge_index.shape[1]
    e_tot = e + n
    loop = jnp.arange(n, dtype=edge_index.dtype)
    src = jnp.concatenate([edge_index[0], loop])
    dst = jnp.concatenate([edge_index[1], loop])
    rows_per_w = pl.cdiv(e_tot, NW * BE)
    ep = NW * rows_per_w * BE
    pad = ep - e_tot
    src2d = jnp.concatenate([src, jnp.zeros((pad,), src.dtype)]).reshape(
        ep // BE, 1, BE)
    dst2d = jnp.concatenate([dst, jnp.zeros((pad,), dst.dtype)]).reshape(
        ep // BE, 1, BE)
    w_cat = jnp.concatenate([W_l, W_r], axis=1)

    xl0, xl1, xr0, xr1 = _matmul_call(x, w_cat, n, in_ch)
    alpha2d, pmax = _alpha_call(xl0, xl1, xr0, xr1, src2d, dst2d, att, e_tot)
    den2 = _denom_call(dst2d, alpha2d, pmax)
    out2 = _agg_call(xl0, xl1, src2d, dst2d, alpha2d, pmax, den2, bias, n)
    return jnp.concatenate([out2[0], out2[1]], axis=1)


# speedup vs baseline: 1.0526x; 1.0200x over previous
import jax
import jax.numpy as jnp
from jax import lax
from jax.experimental import pallas as pl
from jax.experimental.pallas import tpu as pltpu
from jax.experimental.pallas import tpu_sc as plsc

NEG_SLOPE = 0.2
L = 16
NC = 2
NS = 16
NW = NC * NS
BE = 128
BH = 64
H = 128
DR = 88


def _hreduce16(v, op):
    vals = [v[i] for i in range(L)]
    while len(vals) > 1:
        vals = [op(vals[i], vals[i + 1]) for i in range(0, len(vals), 2)]
    return vals[0]


def _gmax_of(pmax_v):
    m = pmax_v[0, 0]
    for i in range(1, NW):
        m = jnp.maximum(m, pmax_v[i, 0])
    return _hreduce16(m, jnp.maximum)


def _matmul_call(x, w_cat, n, in_ch):
    bm = 512
    grid = (pl.cdiv(n, bm),)

    def mm_kernel(x_ref, w_ref, o0, o1, o2, o3):
        res = jax.lax.dot_general(
            x_ref[...], w_ref[...], (((1,), (0,)), ((), ())),
            preferred_element_type=jnp.float32)
        o0[...] = res[:, 0 * H:1 * H]
        o1[...] = res[:, 1 * H:2 * H]
        o2[...] = res[:, 2 * H:3 * H]
        o3[...] = res[:, 3 * H:4 * H]

    return pl.pallas_call(
        mm_kernel,
        grid=grid,
        in_specs=[
            pl.BlockSpec((bm, in_ch), lambda i: (i, 0)),
            pl.BlockSpec((in_ch, 4 * H), lambda i: (0, 0)),
        ],
        out_specs=[pl.BlockSpec((bm, H), lambda i: (i, 0))] * 4,
        out_shape=[jax.ShapeDtypeStruct((n, H), jnp.float32)] * 4,
    )(x, w_cat)


def _alpha_call(xl0, xl1, xr0, xr1, src2d, dst2d, att, e_tot):
    nrows = src2d.shape[0]
    t_per_w = nrows // NW
    hc = H // L

    def body(xl0_h, xl1_h, xr0_h, xr1_h, src_h, dst_h, att_h,
             alpha_h, pmax_h,
             src_v, dst_v, alpha_v, att_v,
             ua0, ua1, va0, va1, ub0, ub1, vb0, vb1, maxv, semA, semB):
        wid = lax.axis_index("s") * NC + lax.axis_index("c")
        rbase = wid * t_per_w
        pltpu.sync_copy(src_h.at[pl.ds(rbase, t_per_w)], src_v)
        pltpu.sync_copy(dst_h.at[pl.ds(rbase, t_per_w)], dst_v)
        pltpu.sync_copy(att_h, att_v)
        lane = lax.iota(jnp.int32, L)
        attv = [att_v[pl.ds(c * L, L)] for c in range(2 * hc)]

        def start(t, half, b0, b1, b2, b3, sem):
            s_sl = src_v.at[t, 0, pl.ds(half * BH, BH)]
            d_sl = dst_v.at[t, 0, pl.ds(half * BH, BH)]
            pltpu.async_copy(xl0_h.at[s_sl], b0, sem)
            pltpu.async_copy(xl1_h.at[s_sl], b1, sem)
            pltpu.async_copy(xr0_h.at[d_sl], b2, sem)
            pltpu.async_copy(xr1_h.at[d_sl], b3, sem)

        def drain(b0, b1, b2, b3, sem):
            pltpu.make_async_copy(xl0_h.at[src_v.at[0, 0, pl.ds(0, BH)]],
                                  b0, sem).wait()
            pltpu.make_async_copy(xl1_h.at[src_v.at[0, 0, pl.ds(0, BH)]],
                                  b1, sem).wait()
            pltpu.make_async_copy(xr0_h.at[dst_v.at[0, 0, pl.ds(0, BH)]],
                                  b2, sem).wait()
            pltpu.make_async_copy(xr1_h.at[dst_v.at[0, 0, pl.ds(0, BH)]],
                                  b3, sem).wait()

        def compute(t, half, u0, u1, v0, v1, mx):
            ebase = (rbase + t) * BE + half * BH

            def eblock(eg, mx2):
                a_vec = jnp.zeros((L,), jnp.float32)
                for j in range(L):
                    e = eg * L + j
                    acc0 = jnp.zeros((L,), jnp.float32)
                    acc1 = jnp.zeros((L,), jnp.float32)
                    for c in range(hc):
                        z = u0[e, pl.ds(c * L, L)] + v0[e, pl.ds(c * L, L)]
                        lr = jnp.maximum(z, NEG_SLOPE * z)
                        if c % 2 == 0:
                            acc0 = acc0 + lr * attv[c]
                        else:
                            acc1 = acc1 + lr * attv[c]
                    for c in range(hc):
                        z = u1[e, pl.ds(c * L, L)] + v1[e, pl.ds(c * L, L)]
                        lr = jnp.maximum(z, NEG_SLOPE * z)
                        if c % 2 == 0:
                            acc0 = acc0 + lr * attv[hc + c]
                        else:
                            acc1 = acc1 + lr * attv[hc + c]
                    sj = _hreduce16(acc0 + acc1, jnp.add)
                    a_vec = jnp.where(lane == j, sj, a_vec)
                gid = ebase + eg * L + lane
                a_vec = jnp.where(gid < e_tot, a_vec, jnp.float32(-1e30))
                alpha_v[t, 0, pl.ds(half * BH + eg * L, L)] = a_vec
                return jnp.maximum(mx2, a_vec)

            return lax.fori_loop(0, BH // L, eblock, mx)

        start(0, 0, ua0, ua1, va0, va1, semA)
        start(0, 1, ub0, ub1, vb0, vb1, semB)

        def rowstep(t, mx):
            drain(ua0, ua1, va0, va1, semA)
            mx = compute(t, 0, ua0, ua1, va0, va1, mx)

            @pl.when(t < t_per_w - 1)
            def _():
                start(t + 1, 0, ua0, ua1, va0, va1, semA)
            drain(ub0, ub1, vb0, vb1, semB)
            mx = compute(t, 1, ub0, ub1, vb0, vb1, mx)

            @pl.when(t < t_per_w - 1)
            def _():
                start(t + 1, 1, ub0, ub1, vb0, vb1, semB)
            return mx

        mvec = lax.fori_loop(0, t_per_w, rowstep,
                             jnp.full((L,), -1e30, jnp.float32))
        maxv[0] = jnp.full((L,), _hreduce16(mvec, jnp.maximum), jnp.float32)
        pltpu.sync_copy(alpha_v, alpha_h.at[pl.ds(rbase, t_per_w)])
        pltpu.sync_copy(maxv, pmax_h.at[wid])

    gb = lambda: pltpu.VMEM((BH, H), jnp.float32)
    fn = pl.kernel(
        body,
        out_type=(jax.ShapeDtypeStruct((nrows, 1, BE), jnp.float32),
                  jax.ShapeDtypeStruct((NW, 1, L), jnp.float32)),
        mesh=plsc.VectorSubcoreMesh(core_axis_name="c", subcore_axis_name="s"),
        scratch_types=[
            pltpu.VMEM((t_per_w, 1, BE), jnp.int32),
            pltpu.VMEM((t_per_w, 1, BE), jnp.int32),
            pltpu.VMEM((t_per_w, 1, BE), jnp.float32),
            pltpu.VMEM((2 * H,), jnp.float32),
            gb(), gb(), gb(), gb(), gb(), gb(), gb(), gb(),
            pltpu.VMEM((1, L), jnp.float32),
            pltpu.SemaphoreType.DMA,
            pltpu.SemaphoreType.DMA,
        ],
    )
    return fn(xl0, xl1, xr0, xr1, src2d, dst2d, att)


def _denom_call(dst2d, alpha2d, pmax):
    nrows = dst2d.shape[0]
    t_per_w = nrows // NW

    def body(dst_h, alpha_h, pmax_h, den_h,
             dst_v, alpha_v, pmax_v, mden, didx, dtmp, den_sh, sem):
        h = lax.axis_index("c")
        tid = lax.axis_index("s")
        wid = tid * NC + h
        rbase = wid * t_per_w
        pltpu.sync_copy(dst_h.at[pl.ds(rbase, t_per_w)], dst_v)
        pltpu.sync_copy(alpha_h.at[pl.ds(rbase, t_per_w)], alpha_v)
        pltpu.sync_copy(pmax_h, pmax_v)
        gmax = _gmax_of(pmax_v)
        lane = lax.iota(jnp.int32, L)
        lanes_c = [lane + c * L for c in range(H // L)]

        def zmb(i, _):
            for c in range(H // L):
                mden[i, pl.ds(c * L, L)] = jnp.zeros((L,), jnp.float32)
            return 0
        lax.fori_loop(0, BE, zmb, 0)

        @pl.when(tid < DR // 8)
        def _():
            pltpu.sync_copy(mden.at[pl.ds(0, 8)],
                            den_sh.at[pl.ds(tid * 8, 8)])
        plsc.subcore_barrier()

        def step(t, _):
            def egroup(k, _):
                a = alpha_v[t, 0, pl.ds(k * L, L)]
                wvec = jnp.exp(a - gmax)
                dvec = dst_v[t, 0, pl.ds(k * L, L)]
                dmod = jnp.bitwise_and(dvec, jnp.int32(H - 1))
                didx[0, pl.ds(k * L, L)] = jnp.right_shift(dvec, 7)
                e0 = k * L
                for j in range(L):
                    w = wvec[j]
                    dm = dmod[j]
                    for c in range(H // L):
                        mden[e0 + j, pl.ds(c * L, L)] = jnp.where(
                            lanes_c[c] == dm, w, jnp.float32(0.0))
                return 0
            lax.fori_loop(0, BE // L, egroup, 0)
            pltpu.sync_copy(mden, den_sh.at[didx.at[0]], add=True)
            return 0
        lax.fori_loop(0, t_per_w, step, 0)
        plsc.subcore_barrier()

        @pl.when(tid < DR // 8)
        def _():
            pltpu.sync_copy(den_sh.at[pl.ds(tid * 8, 8)], dtmp)
            pltpu.sync_copy(dtmp, den_h.at[h, pl.ds(tid * 8, 8)])

    fn = pl.kernel(
        body,
        out_type=jax.ShapeDtypeStruct((NC, DR, H), jnp.float32),
        mesh=plsc.VectorSubcoreMesh(core_axis_name="c", subcore_axis_name="s"),
        scratch_types=[
            pltpu.VMEM((t_per_w, 1, BE), jnp.int32),
            pltpu.VMEM((t_per_w, 1, BE), jnp.float32),
            pltpu.VMEM((NW, 1, L), jnp.float32),
            pltpu.VMEM((BE, H), jnp.float32),
            pltpu.VMEM((1, BE), jnp.int32),
            pltpu.VMEM((8, H), jnp.float32),
            pltpu.VMEM_SHARED((DR, H), jnp.float32),
            pltpu.SemaphoreType.DMA,
        ],
    )
    return fn(dst2d, alpha2d, pmax)


def _agg_call(xl0, xl1, src2d, dst2d, alpha2d, pmax, den2, bias, n):
    nrows = src2d.shape[0]
    t_rows = nrows // NS
    pairs = t_rows // 2
    npad = ((n + NS * L - 1) // (NS * L)) * (NS * L)
    npt = npad // NS
    hc = H // L
    drows = npt // H

    def body(xl0_h, xl1_h, src_h, dst_h, alpha_h, pmax_h, den_h, bias_h,
             out_h,
             src_s, dst_s, alpha_s, src_b, dst_b, alpha_b, pmax_v, bias_v,
             gA, gB, mA, mB, out_sh, gsA, gsB, ssA, ssB):
        h = lax.axis_index("c")
        tid = lax.axis_index("s")
        rbase = tid * t_rows
        pltpu.sync_copy(pmax_h, pmax_v)
        pltpu.sync_copy(bias_h, bias_v)
        gmax = _gmax_of(pmax_v)

        def zmb(i, _):
            for c in range(hc):
                mA[i, pl.ds(c * L, L)] = jnp.zeros((L,), jnp.float32)
            return 0
        lax.fori_loop(0, BH, zmb, 0)
        n0 = tid * npt
        for b in range(npt // BH):
            pltpu.sync_copy(mA, out_sh.at[pl.ds(n0 + b * BH, BH)])
        plsc.subcore_barrier()

        def load_idx(t, s_r, d_r, a_r):
            pltpu.sync_copy(src_h.at[pl.ds(rbase + t, 1)], s_r)
            pltpu.sync_copy(dst_h.at[pl.ds(rbase + t, 1)], d_r)
            pltpu.sync_copy(alpha_h.at[pl.ds(rbase + t, 1)], a_r)

        def startg(s_r, half, buf, sem):
            s_sl = s_r.at[0, 0, pl.ds(half * BH, BH)]

            @pl.when(h == 0)
            def _():
                pltpu.async_copy(xl0_h.at[s_sl], buf, sem)

            @pl.when(h == 1)
            def _():
                pltpu.async_copy(xl1_h.at[s_sl], buf, sem)

        def draing(buf, sem):
            pltpu.make_async_copy(
                xl0_h.at[src_s.at[0, 0, pl.ds(0, BH)]], buf, sem).wait()

        def fill(a_r, half, gbuf, mb):
            def egroup(k, _):
                a = a_r[0, 0, pl.ds(half * BH + k * L, L)]
                wvec = jnp.exp(a - gmax)
                e0 = k * L
                for j in range(L):
                    w = wvec[j]
                    for c in range(hc):
                        mb[e0 + j, pl.ds(c * L, L)] = (
                            gb_read(gbuf, e0 + j, c) * w)
                return 0
            lax.fori_loop(0, BH // L, egroup, 0)

        def gb_read(gbuf, e, c):
            return gbuf[e, pl.ds(c * L, L)]

        def starts(d_r, half, mb, sem):
            d_sl = d_r.at[0, 0, pl.ds(half * BH, BH)]
            return pltpu.async_copy(mb, out_sh.at[d_sl], sem, add=True)

        load_idx(0, src_s, dst_s, alpha_s)
        startg(src_s, 0, gA, gsA)

        def pairstep(p, _):
            load_idx(2 * p + 1, src_b, dst_b, alpha_b)
            draing(gA, gsA)
            startg(src_s, 1, gB, gsB)
            fill(alpha_s, 0, gA, mA)
            d0 = starts(dst_s, 0, mA, ssA)
            draing(gB, gsB)
            startg(src_b, 0, gA, gsA)
            fill(alpha_s, 1, gB, mB)
            d1 = starts(dst_s, 1, mB, ssB)
            draing(gA, gsA)
            startg(src_b, 1, gB, gsB)
            d0.wait()
            fill(alpha_b, 0, gA, mA)
            d2 = starts(dst_b, 0, mA, ssA)
            draing(gB, gsB)
            d1.wait()

            @pl.when(p < pairs - 1)
            def _():
                load_idx(2 * p + 2, src_s, dst_s, alpha_s)
                startg(src_s, 0, gA, gsA)
            fill(alpha_b, 1, gB, mB)
            d3 = starts(dst_b, 1, mB, ssB)
            d2.wait()
            d3.wait()
            return 0
        lax.fori_loop(0, pairs, pairstep, 0)
        plsc.subcore_barrier()

        r_lo = tid * drows
        w0 = (r_lo // 8) * 8
        pltpu.sync_copy(den_h.at[0, pl.ds(w0, 2 * 8)], gB.at[pl.ds(0, 16)])
        pltpu.sync_copy(den_h.at[1, pl.ds(w0, 2 * 8)], gB.at[pl.ds(16, 16)])
        roff = r_lo - w0

        nvalid = jnp.maximum(0, jnp.minimum(npt, n - n0))
        nblk = nvalid // L
        bbs = [bias_v[pl.ds(h * H + c * L, L)] for c in range(hc)]

        def blk(b, _):
            r0 = n0 + b * L
            pltpu.sync_copy(out_sh.at[pl.ds(r0, L)], gA.at[pl.ds(0, L)])
            drow = roff + b // hc
            dsl = pl.ds((b % hc) * L, L)
            dv = gB[drow, dsl] + gB[16 + drow, dsl]
            ivec = jnp.float32(1.0) / (dv + jnp.float32(1e-16))
            for j in range(L):
                sj = ivec[j]
                for c in range(hc):
                    mA[j, pl.ds(c * L, L)] = (
                        gA[j, pl.ds(c * L, L)] * sj + bbs[c])
            pltpu.sync_copy(mA.at[pl.ds(0, L)], out_h.at[h, pl.ds(r0, L)])
            return 0
        lax.fori_loop(0, nblk, blk, 0)

    fn = pl.kernel(
        body,
        out_type=jax.ShapeDtypeStruct((NC, n, H), jnp.float32),
        mesh=plsc.VectorSubcoreMesh(core_axis_name="c", subcore_axis_name="s"),
        scratch_types=[
            pltpu.VMEM((1, 1, BE), jnp.int32),
            pltpu.VMEM((1, 1, BE), jnp.int32),
            pltpu.VMEM((1, 1, BE), jnp.float32),
            pltpu.VMEM((1, 1, BE), jnp.int32),
            pltpu.VMEM((1, 1, BE), jnp.int32),
            pltpu.VMEM((1, 1, BE), jnp.float32),
            pltpu.VMEM((NW, 1, L), jnp.float32),
            pltpu.VMEM((2 * H,), jnp.float32),
            pltpu.VMEM((BH, H), jnp.float32),
            pltpu.VMEM((BH, H), jnp.float32),
            pltpu.VMEM((BH, H), jnp.float32),
            pltpu.VMEM((BH, H), jnp.float32),
            pltpu.VMEM_SHARED((npad, H), jnp.float32),
            pltpu.SemaphoreType.DMA,
            pltpu.SemaphoreType.DMA,
            pltpu.SemaphoreType.DMA,
            pltpu.SemaphoreType.DMA,
        ],
    )
    return fn(xl0, xl1, src2d, dst2d, alpha2d, pmax, den2, bias)


def kernel(x, edge_index, W_l, W_r, att, bias):
    n, in_ch = x.shape
    e = edge_index.shape[1]
    e_tot = e + n
    loop = jnp.arange(n, dtype=edge_index.dtype)
    src = jnp.concatenate([edge_index[0], loop])
    dst = jnp.concatenate([edge_index[1], loop])
    rows_per_w = pl.cdiv(e_tot, NW * BE)
    ep = NW * rows_per_w * BE
    pad = ep - e_tot
    src2d = jnp.concatenate([src, jnp.zeros((pad,), src.dtype)]).reshape(
        ep // BE, 1, BE)
    dst2d = jnp.concatenate([dst, jnp.zeros((pad,), dst.dtype)]).reshape(
        ep // BE, 1, BE)
    w_cat = jnp.concatenate([W_l, W_r], axis=1)

    xl0, xl1, xr0, xr1 = _matmul_call(x, w_cat, n, in_ch)
    alpha2d, pmax = _alpha_call(xl0, xl1, xr0, xr1, src2d, dst2d, att, e_tot)
    den2 = _denom_call(dst2d, alpha2d, pmax)
    out2 = _agg_call(xl0, xl1, src2d, dst2d, alpha2d, pmax, den2, bias, n)
    return jnp.concatenate([out2[0], out2[1]], axis=1)
